# baseline (speedup 1.0000x reference)
import functools

import jax
import jax.numpy as jnp
from jax import lax
from jax.experimental import pallas as pl
from jax.experimental.pallas import tpu as pltpu
from jax.experimental.pallas import tpu_sc as plsc

B = 128
N = 32768
KP = 1639
KB = 104896
NC = 2
NS = 16
NW = NC * NS
RPW = B // NW
RPS = B // NS
KEEP = 2048
CAND = 16384
IMIN = -2147483648
M31 = 0x7FFFFFFF

_i32 = jnp.int32
_f32 = jnp.float32


def _scal(v):
    return v[0] if getattr(v, "ndim", 0) else v


def _sortable(v):
    bits = lax.bitcast_convert_type(v, _i32)
    return bits ^ ((bits >> 31) & M31)


def _crossing(vec, k, lane):
    r = jnp.flip(vec, 0)
    cum = plsc.cumsum(r)
    m = cum >= k
    j0 = _scal(plsc.all_reduce_ffs(m))
    sel = lane == j0
    cum_at = jnp.sum(jnp.where(sel, cum, 0))
    r_at = jnp.sum(jnp.where(sel, r, 0))
    u = 15 - j0
    k_new = k - (cum_at - r_at)
    return u, k_new, r_at


def _analyze_hist(hist_v, k, lane):
    zero16 = jnp.zeros((16,), _i32)

    def chunk_body(cj, cv):
        def inner(j, acc):
            return acc + hist_v[pl.ds(cj * 256 + j * 16, 16)]
        acc = lax.fori_loop(0, 16, inner, zero16)
        return jnp.where(lane == cj, jnp.sum(acc), cv)

    chunkvec = lax.fori_loop(0, 16, chunk_body, zero16)
    cstar, kc, _ = _crossing(chunkvec, k, lane)

    def bin_body(j, bv):
        w = hist_v[pl.ds(cstar * 256 + j * 16, 16)]
        return jnp.where(lane == j, jnp.sum(w), bv)

    binvec = lax.fori_loop(0, 16, bin_body, zero16)
    b_in, k_new, _ = _crossing(binvec, kc, lane)

    def zbody(j, _):
        hist_v[pl.ds(j * 16, 16)] = zero16
        return 0

    lax.fori_loop(0, 256, zbody, 0)
    return cstar * 16 + b_in, k_new


def _pc(mask):
    return _scal(plsc.all_reduce_population_count(mask))


def _compact(dst_ref, y, mask, off_v):
    pos = plsc.cumsum(jnp.where(mask, 1, 0))
    plsc.store_scatter(dst_ref, [off_v + pos - 1], y, mask=mask)
    return off_v + plsc.all_reduce_population_count(mask)


def _k1_body(x_hbm, kept_hbm, t_hbm, xrow_v, y_v, hist_v, cand_v, keep_v, tb_v):
    c = lax.axis_index("c")
    s = lax.axis_index("s")
    wid = s * NC + c
    lane = lax.iota(_i32, 16)
    zero16 = jnp.zeros((16,), _i32)
    ones16 = jnp.ones((16,), _i32)
    fill16 = jnp.full((16,), IMIN, _i32)

    def zh(j, _):
        hist_v[pl.ds(j * 16, 16)] = zero16
        return 0

    lax.fori_loop(0, 256, zh, 0)

    def row_body(r, t_vec):
        row = wid * RPW + r
        pltpu.sync_copy(x_hbm.at[row], xrow_v)

        def s1(i, _):
            for u in range(4):
                off = (i * 4 + u) * 16
                y = _sortable(xrow_v[pl.ds(off, 16)])
                y_v[pl.ds(off, 16)] = y
                b = ((y >> 24) + 128) & 0xFF
                plsc.addupdate_scatter(hist_v, [b * 16 + lane], ones16)
            return 0

        lax.fori_loop(0, N // 64, s1, 0)
        b0, k1r = _analyze_hist(hist_v, _i32(KP), lane)

        def zk(j, _):
            keep_v[pl.ds(j * 16, 16)] = fill16
            return 0

        lax.fori_loop(0, KEEP // 16, zk, 0)

        b0_v = zero16 + b0

        def s2(i, carry):
            ko, co = carry
            for u in range(4):
                off = (i * 4 + u) * 16
                y = y_v[pl.ds(off, 16)]
                b = ((y >> 24) + 128) & 0xFF
                mk = (b > b0_v) & (ko < KEEP - 15)
                ko = _compact(keep_v, y, mk, ko)
                mc = (b == b0_v) & (co < CAND - 15)
                idx = ((y >> 16) & 0xFF) * 16 + lane
                plsc.addupdate_scatter(hist_v, [idx], ones16, mask=mc)
                co = _compact(cand_v, y, mc, co)
            return ko, co

        ko_v, n_v = lax.fori_loop(0, N // 64, s2, (zero16, zero16))
        t_p = (b0 - 128) << 24

        def lvl(l, carry):
            ko, n_v, k_rem, t_p = carry
            sh = 24 - 8 * l
            b_l, k_new = _analyze_hist(hist_v, k_rem, lane)
            b_l_v = zero16 + b_l

            def scn(i, cy):
                ko2, co2 = cy
                off = i * 16
                valid = (off + lane) < n_v
                y = cand_v[pl.ds(off, 16)]
                byt = (y >> sh) & 0xFF
                mk = valid & (byt > b_l_v) & (ko2 < KEEP - 15)
                ko2 = _compact(keep_v, y, mk, ko2)
                mc = valid & (byt == b_l_v)
                idx = ((y >> (sh - 8)) & 0xFF) * 16 + lane
                plsc.addupdate_scatter(hist_v, [idx], ones16, mask=mc)
                co2 = _compact(cand_v, y, mc, co2)
                return ko2, co2

            ko, n2_v = lax.fori_loop(0, (n_v[0] + 15) >> 4, scn, (ko, zero16))
            t_p = t_p | (b_l << sh)
            return ko, n2_v, k_new, t_p

        ko_v, n3_v, k3, t_p = lax.fori_loop(1, 3, lvl, (ko_v, n_v, k1r, t_p))

        b3, _ = _analyze_hist(hist_v, k3, lane)
        t_row = t_p | b3
        b3_v = zero16 + b3

        def fin(i, ko2):
            off = i * 16
            valid = (off + lane) < n3_v
            y = cand_v[pl.ds(off, 16)]
            mk = valid & ((y & 0xFF) >= b3_v) & (ko2 < KEEP - 15)
            return _compact(keep_v, y, mk, ko2)

        lax.fori_loop(0, (n3_v[0] + 15) >> 4, fin, ko_v)
        pltpu.sync_copy(keep_v, kept_hbm.at[row])
        return jnp.where(lane == r, t_row, t_vec)

    t_vec = lax.fori_loop(0, RPW, row_body, jnp.zeros((16,), _i32))
    tb_v[...] = t_vec
    pltpu.sync_copy(tb_v, t_hbm.at[pl.ds(wid * 16, 16)])


def _share_analyze(k, s, lane, hist_v, tmp_v, st_v, cnt_v, sh_hist, sh_cnt):
    zero16 = jnp.zeros((16,), _i32)
    pltpu.sync_copy(hist_v, sh_hist.at[s])

    def zh(j, _):
        hist_v[pl.ds(j * 16, 16)] = zero16
        return 0

    lax.fori_loop(0, 256, zh, 0)
    plsc.subcore_barrier()

    def wbody(w, accs):
        pltpu.sync_copy(sh_hist.at[w, pl.ds(s * 256, 256)], tmp_v)
        return tuple(accs[j] + tmp_v[pl.ds(j * 16, 16)] for j in range(16))

    accs = lax.fori_loop(0, 16, wbody, (zero16,) * 16)
    binvec = zero16
    for j in range(16):
        binvec = jnp.where(lane == j, jnp.sum(accs[j]), binvec)
    st_v[...] = binvec
    pltpu.sync_copy(st_v, sh_cnt.at[pl.ds(s * 16, 16)])
    plsc.subcore_barrier()
    pltpu.sync_copy(sh_cnt, cnt_v)

    def cb(cj, cv):
        return jnp.where(lane == cj, jnp.sum(cnt_v[pl.ds(cj * 16, 16)]), cv)

    chunkvec = lax.fori_loop(0, 16, cb, zero16)
    cstar, kc, _ = _crossing(chunkvec, k, lane)
    binvec2 = cnt_v[pl.ds(cstar * 16, 16)]
    b_in, k_new, _ = _crossing(binvec2, kc, lane)
    return cstar * 16 + b_in, k_new


def _k2_body(x_hbm, kept_hbm, t_hbm, out_hbm,
             kv_v, xrow_v, hist_v, cand_v, tmp_v, st_v, cnt_v, t16_v,
             sh_hist, sh_cnt):
    c = lax.axis_index("c")
    s = lax.axis_index("s")
    wid = s * NC + c
    lane = lax.iota(_i32, 16)
    zero16 = jnp.zeros((16,), _i32)
    ones16 = jnp.ones((16,), _i32)
    NKV = RPS * KEEP

    def zh(j, _):
        hist_v[pl.ds(j * 16, 16)] = zero16
        return 0

    lax.fori_loop(0, 256, zh, 0)
    pltpu.sync_copy(kept_hbm.at[pl.ds(s * NKV, NKV)], kv_v)

    def h0(i, _):
        for u in range(4):
            off = (i * 4 + u) * 16
            y = kv_v[pl.ds(off, 16)]
            b = ((y >> 24) + 128) & 0xFF
            plsc.addupdate_scatter(hist_v, [b * 16 + lane], ones16)
        return 0

    lax.fori_loop(0, NKV // 64, h0, 0)
    b0, k = _share_analyze(_i32(KB), s, lane, hist_v, tmp_v, st_v, cnt_v,
                           sh_hist, sh_cnt)

    b0_v = zero16 + b0

    def c0(i, co):
        for u in range(4):
            off = (i * 4 + u) * 16
            y = kv_v[pl.ds(off, 16)]
            b = ((y >> 24) + 128) & 0xFF
            mc = (b == b0_v) & (co < CAND - 15)
            idx = ((y >> 16) & 0xFF) * 16 + lane
            plsc.addupdate_scatter(hist_v, [idx], ones16, mask=mc)
            co = _compact(cand_v, y, mc, co)
        return co

    n_v = lax.fori_loop(0, NKV // 64, c0, zero16)
    T = (b0 - 128) << 24

    for l in (1, 2):
        sh = 24 - 8 * l
        b_l, k = _share_analyze(k, s, lane, hist_v, tmp_v, st_v, cnt_v,
                                sh_hist, sh_cnt)

        b_l_v = zero16 + b_l

        def cl(i, co, n_v=n_v, b_l_v=b_l_v, sh=sh):
            off = i * 16
            valid = (off + lane) < n_v
            y = cand_v[pl.ds(off, 16)]
            byt = (y >> sh) & 0xFF
            mc = valid & (byt == b_l_v) & (co < CAND - 15)
            idx = ((y >> (sh - 8)) & 0xFF) * 16 + lane
            plsc.addupdate_scatter(hist_v, [idx], ones16, mask=mc)
            return _compact(cand_v, y, mc, co)

        n_v = lax.fori_loop(0, (n_v[0] + 15) >> 4, cl, zero16)
        T = T | (b_l << sh)

    b3, k = _share_analyze(k, s, lane, hist_v, tmp_v, st_v, cnt_v,
                           sh_hist, sh_cnt)
    T = T | b3

    pltpu.sync_copy(t_hbm.at[pl.ds(wid * 16, 16)], t16_v)
    thr_vec = jnp.maximum(t16_v[...], T)

    def mrow(r, _):
        row = wid * RPW + r
        pltpu.sync_copy(x_hbm.at[row], xrow_v)
        thr = jnp.max(jnp.where(lane == r, thr_vec, IMIN))

        def mb(i, _):
            for u in range(4):
                off = (i * 4 + u) * 16
                v = xrow_v[pl.ds(off, 16)]
                keep = _sortable(v) >= thr
                xrow_v[pl.ds(off, 16)] = jnp.where(keep, v, _f32(0.0))
            return 0

        lax.fori_loop(0, N // 64, mb, 0)
        pltpu.sync_copy(xrow_v, out_hbm.at[row])
        return 0

    lax.fori_loop(0, RPW, mrow, 0)


@functools.cache
def _build():
    mesh = plsc.VectorSubcoreMesh(core_axis_name="c", subcore_axis_name="s")
    params = pltpu.CompilerParams(needs_layout_passes=False)
    k1 = pl.kernel(
        _k1_body,
        out_type=(
            jax.ShapeDtypeStruct((B, KEEP), _i32),
            jax.ShapeDtypeStruct((NW * 16,), _i32),
        ),
        mesh=mesh,
        scratch_types=[
            pltpu.VMEM((N,), _f32),
            pltpu.VMEM((N,), _i32),
            pltpu.VMEM((4096,), _i32),
            pltpu.VMEM((CAND,), _i32),
            pltpu.VMEM((KEEP,), _i32),
            pltpu.VMEM((16,), _i32),
        ],
        compiler_params=params,
    )
    k2 = pl.kernel(
        _k2_body,
        out_type=jax.ShapeDtypeStruct((B, N), _f32),
        mesh=mesh,
        scratch_types=[
            pltpu.VMEM((RPS * KEEP,), _i32),
            pltpu.VMEM((N,), _f32),
            pltpu.VMEM((4096,), _i32),
            pltpu.VMEM((CAND,), _i32),
            pltpu.VMEM((256,), _i32),
            pltpu.VMEM((16,), _i32),
            pltpu.VMEM((256,), _i32),
            pltpu.VMEM((16,), _i32),
            pltpu.VMEM_SHARED((16, 4096), _i32),
            pltpu.VMEM_SHARED((256,), _i32),
        ],
        compiler_params=params,
    )
    return k1, k2


def kernel(x):
    k1, k2 = _build()
    kept, t = k1(x)
    return k2(x, kept.reshape(-1), t)


# speedup vs baseline: 1.0929x; 1.0929x over previous
import functools

import jax
import jax.numpy as jnp
from jax import lax
from jax.experimental import pallas as pl
from jax.experimental.pallas import tpu as pltpu
from jax.experimental.pallas import tpu_sc as plsc

B = 128
N = 32768
KP = 1639
KB = 104896
NC = 2
NS = 16
NW = NC * NS
RPW = B // NW
RPS = B // NS
KEEP = 2048
CAND = 16384
IMIN = -2147483648
M31 = 0x7FFFFFFF

_i32 = jnp.int32
_f32 = jnp.float32


def _scal(v):
    return v[0] if getattr(v, "ndim", 0) else v


def _sortable(v):
    bits = lax.bitcast_convert_type(v, _i32)
    return bits ^ ((bits >> 31) & M31)


def _crossing(vec, k, lane):
    r = jnp.flip(vec, 0)
    cum = plsc.cumsum(r)
    m = cum >= k
    j0 = _scal(plsc.all_reduce_ffs(m))
    sel = lane == j0
    cum_at = jnp.sum(jnp.where(sel, cum, 0))
    r_at = jnp.sum(jnp.where(sel, r, 0))
    u = 15 - j0
    k_new = k - (cum_at - r_at)
    return u, k_new, r_at


def _analyze_hist(hist_v, k, lane, base=0):
    zero16 = jnp.zeros((16,), _i32)

    def chunk_body(cj, cv):
        def inner(j, acc):
            return acc + hist_v[pl.ds(base + cj * 256 + j * 16, 16)]
        acc = lax.fori_loop(0, 16, inner, zero16)
        return jnp.where(lane == cj, jnp.sum(acc), cv)

    chunkvec = lax.fori_loop(0, 16, chunk_body, zero16)
    cstar, kc, _ = _crossing(chunkvec, k, lane)

    def bin_body(j, bv):
        w = hist_v[pl.ds(base + cstar * 256 + j * 16, 16)]
        return jnp.where(lane == j, jnp.sum(w), bv)

    binvec = lax.fori_loop(0, 16, bin_body, zero16)
    b_in, k_new, _ = _crossing(binvec, kc, lane)

    def zbody(j, _):
        hist_v[pl.ds(base + j * 16, 16)] = zero16
        return 0

    lax.fori_loop(0, 256, zbody, 0)
    return cstar * 16 + b_in, k_new


def _pc(mask):
    return _scal(plsc.all_reduce_population_count(mask))


def _compact(dst_ref, y, mask, off_v):
    pos = plsc.cumsum(jnp.where(mask, 1, 0))
    plsc.store_scatter(dst_ref, [off_v + pos - 1], y, mask=mask)
    return off_v + plsc.all_reduce_population_count(mask)


def _k1_body(x_hbm, kept_hbm, t_hbm, xrow_v, y_v, hist_v, cand_v, keep_v, tb_v):
    c = lax.axis_index("c")
    s = lax.axis_index("s")
    wid = s * NC + c
    lane = lax.iota(_i32, 16)
    zero16 = jnp.zeros((16,), _i32)
    ones16 = jnp.ones((16,), _i32)
    fill16 = jnp.full((16,), IMIN, _i32)

    def zh(j, _):
        hist_v[pl.ds(j * 16, 16)] = zero16
        return 0

    lax.fori_loop(0, 512, zh, 0)

    def row_body(r, carry):
        t_vec, bg = carry
        row = wid * RPW + r
        pltpu.sync_copy(x_hbm.at[row], xrow_v)

        def zk(j, _):
            keep_v[pl.ds(j * 16, 16)] = fill16
            return 0

        lax.fori_loop(0, KEEP // 16, zk, 0)
        bg_v = zero16 + bg

        def s1(i, carry):
            ko, co = carry
            for u in range(4):
                off = (i * 4 + u) * 16
                y = _sortable(xrow_v[pl.ds(off, 16)])
                y_v[pl.ds(off, 16)] = y
                b = ((y >> 24) + 128) & 0xFF
                plsc.addupdate_scatter(hist_v, [b * 16 + lane], ones16)
                mk = (b > bg_v) & (ko < KEEP - 15)
                ko = _compact(keep_v, y, mk, ko)
                mc = (b == bg_v) & (co < CAND - 15)
                idx = 4096 + ((y >> 16) & 0xFF) * 16 + lane
                plsc.addupdate_scatter(hist_v, [idx], ones16, mask=mc)
                co = _compact(cand_v, y, mc, co)
            return ko, co

        ko_v, n_v = lax.fori_loop(0, N // 64, s1, (zero16, zero16))
        b0, k1r = _analyze_hist(hist_v, _i32(KP), lane)

        miss = (b0 != bg).astype(_i32)
        b0_v = zero16 + b0

        def zk2(j, _):
            keep_v[pl.ds(j * 16, 16)] = fill16
            return 0

        lax.fori_loop(0, miss * (KEEP // 16), zk2, 0)

        def zh2(j, _):
            hist_v[pl.ds(4096 + j * 16, 16)] = zero16
            return 0

        lax.fori_loop(0, miss * 256, zh2, 0)

        def s2(i, carry):
            ko, co = carry
            for u in range(4):
                off = (i * 4 + u) * 16
                y = y_v[pl.ds(off, 16)]
                b = ((y >> 24) + 128) & 0xFF
                mk = (b > b0_v) & (ko < KEEP - 15)
                ko = _compact(keep_v, y, mk, ko)
                mc = (b == b0_v) & (co < CAND - 15)
                idx = 4096 + ((y >> 16) & 0xFF) * 16 + lane
                plsc.addupdate_scatter(hist_v, [idx], ones16, mask=mc)
                co = _compact(cand_v, y, mc, co)
            return ko, co

        ko_r, n_r = lax.fori_loop(0, miss * (N // 64), s2, (zero16, zero16))
        ko_v = jnp.where(miss > 0, ko_r, ko_v)
        n_v = jnp.where(miss > 0, n_r, n_v)
        t_p = (b0 - 128) << 24

        def lvl(l, carry):
            ko, n_v, k_rem, t_p = carry
            sh = 24 - 8 * l
            b_l, k_new = _analyze_hist(hist_v, k_rem, lane, base=4096 * (l & 1))
            b_l_v = zero16 + b_l
            obase = 4096 * ((l + 1) & 1)

            def scn(i, cy):
                ko2, co2 = cy
                off = i * 16
                valid = (off + lane) < n_v
                y = cand_v[pl.ds(off, 16)]
                byt = (y >> sh) & 0xFF
                mk = valid & (byt > b_l_v) & (ko2 < KEEP - 15)
                ko2 = _compact(keep_v, y, mk, ko2)
                mc = valid & (byt == b_l_v)
                idx = obase + ((y >> (sh - 8)) & 0xFF) * 16 + lane
                plsc.addupdate_scatter(hist_v, [idx], ones16, mask=mc)
                co2 = _compact(cand_v, y, mc, co2)
                return ko2, co2

            ko, n2_v = lax.fori_loop(0, (n_v[0] + 15) >> 4, scn, (ko, zero16))
            t_p = t_p | (b_l << sh)
            return ko, n2_v, k_new, t_p

        ko_v, n3_v, k3, t_p = lax.fori_loop(1, 3, lvl, (ko_v, n_v, k1r, t_p))

        b3, _ = _analyze_hist(hist_v, k3, lane, base=4096)
        t_row = t_p | b3
        b3_v = zero16 + b3

        def fin(i, ko2):
            off = i * 16
            valid = (off + lane) < n3_v
            y = cand_v[pl.ds(off, 16)]
            mk = valid & ((y & 0xFF) >= b3_v) & (ko2 < KEEP - 15)
            return _compact(keep_v, y, mk, ko2)

        lax.fori_loop(0, (n3_v[0] + 15) >> 4, fin, ko_v)
        pltpu.sync_copy(keep_v, kept_hbm.at[row])
        return jnp.where(lane == r, t_row, t_vec), b0

    t_vec, _ = lax.fori_loop(0, RPW, row_body,
                             (jnp.zeros((16,), _i32), _i32(-1)))
    tb_v[...] = t_vec
    pltpu.sync_copy(tb_v, t_hbm.at[pl.ds(wid * 16, 16)])


def _share_analyze(k, s, lane, hist_v, tmp_v, st_v, cnt_v, sh_hist, sh_cnt):
    zero16 = jnp.zeros((16,), _i32)
    pltpu.sync_copy(hist_v, sh_hist.at[s])

    def zh(j, _):
        hist_v[pl.ds(j * 16, 16)] = zero16
        return 0

    lax.fori_loop(0, 256, zh, 0)
    plsc.subcore_barrier()

    def wbody(w, accs):
        pltpu.sync_copy(sh_hist.at[w, pl.ds(s * 256, 256)], tmp_v)
        return tuple(accs[j] + tmp_v[pl.ds(j * 16, 16)] for j in range(16))

    accs = lax.fori_loop(0, 16, wbody, (zero16,) * 16)
    binvec = zero16
    for j in range(16):
        binvec = jnp.where(lane == j, jnp.sum(accs[j]), binvec)
    st_v[...] = binvec
    pltpu.sync_copy(st_v, sh_cnt.at[pl.ds(s * 16, 16)])
    plsc.subcore_barrier()
    pltpu.sync_copy(sh_cnt, cnt_v)

    def cb(cj, cv):
        return jnp.where(lane == cj, jnp.sum(cnt_v[pl.ds(cj * 16, 16)]), cv)

    chunkvec = lax.fori_loop(0, 16, cb, zero16)
    cstar, kc, _ = _crossing(chunkvec, k, lane)
    binvec2 = cnt_v[pl.ds(cstar * 16, 16)]
    b_in, k_new, _ = _crossing(binvec2, kc, lane)
    return cstar * 16 + b_in, k_new


def _k2_body(x_hbm, kept_hbm, t_hbm, out_hbm,
             kv_v, xrow_v, hist_v, cand_v, tmp_v, st_v, cnt_v, t16_v,
             sh_hist, sh_cnt):
    c = lax.axis_index("c")
    s = lax.axis_index("s")
    wid = s * NC + c
    lane = lax.iota(_i32, 16)
    zero16 = jnp.zeros((16,), _i32)
    ones16 = jnp.ones((16,), _i32)
    NKV = RPS * KEEP

    def zh(j, _):
        hist_v[pl.ds(j * 16, 16)] = zero16
        return 0

    lax.fori_loop(0, 256, zh, 0)
    pltpu.sync_copy(kept_hbm.at[pl.ds(s * NKV, NKV)], kv_v)

    def h0(i, _):
        for u in range(4):
            off = (i * 4 + u) * 16
            y = kv_v[pl.ds(off, 16)]
            b = ((y >> 24) + 128) & 0xFF
            plsc.addupdate_scatter(hist_v, [b * 16 + lane], ones16)
        return 0

    lax.fori_loop(0, NKV // 64, h0, 0)
    b0, k = _share_analyze(_i32(KB), s, lane, hist_v, tmp_v, st_v, cnt_v,
                           sh_hist, sh_cnt)

    b0_v = zero16 + b0

    def c0(i, co):
        for u in range(4):
            off = (i * 4 + u) * 16
            y = kv_v[pl.ds(off, 16)]
            b = ((y >> 24) + 128) & 0xFF
            mc = (b == b0_v) & (co < CAND - 15)
            idx = ((y >> 16) & 0xFF) * 16 + lane
            plsc.addupdate_scatter(hist_v, [idx], ones16, mask=mc)
            co = _compact(cand_v, y, mc, co)
        return co

    n_v = lax.fori_loop(0, NKV // 64, c0, zero16)
    T = (b0 - 128) << 24

    for l in (1, 2):
        sh = 24 - 8 * l
        b_l, k = _share_analyze(k, s, lane, hist_v, tmp_v, st_v, cnt_v,
                                sh_hist, sh_cnt)

        b_l_v = zero16 + b_l

        def cl(i, co, n_v=n_v, b_l_v=b_l_v, sh=sh):
            off = i * 16
            valid = (off + lane) < n_v
            y = cand_v[pl.ds(off, 16)]
            byt = (y >> sh) & 0xFF
            mc = valid & (byt == b_l_v) & (co < CAND - 15)
            idx = ((y >> (sh - 8)) & 0xFF) * 16 + lane
            plsc.addupdate_scatter(hist_v, [idx], ones16, mask=mc)
            return _compact(cand_v, y, mc, co)

        n_v = lax.fori_loop(0, (n_v[0] + 15) >> 4, cl, zero16)
        T = T | (b_l << sh)

    b3, k = _share_analyze(k, s, lane, hist_v, tmp_v, st_v, cnt_v,
                           sh_hist, sh_cnt)
    T = T | b3

    pltpu.sync_copy(t_hbm.at[pl.ds(wid * 16, 16)], t16_v)
    thr_vec = jnp.maximum(t16_v[...], T)

    def mrow(r, _):
        row = wid * RPW + r
        pltpu.sync_copy(x_hbm.at[row], xrow_v)
        thr = jnp.max(jnp.where(lane == r, thr_vec, IMIN))

        def mb(i, _):
            for u in range(4):
                off = (i * 4 + u) * 16
                v = xrow_v[pl.ds(off, 16)]
                keep = _sortable(v) >= thr
                xrow_v[pl.ds(off, 16)] = jnp.where(keep, v, _f32(0.0))
            return 0

        lax.fori_loop(0, N // 64, mb, 0)
        pltpu.sync_copy(xrow_v, out_hbm.at[row])
        return 0

    lax.fori_loop(0, RPW, mrow, 0)


@functools.cache
def _build():
    mesh = plsc.VectorSubcoreMesh(core_axis_name="c", subcore_axis_name="s")
    params = pltpu.CompilerParams(needs_layout_passes=False)
    k1 = pl.kernel(
        _k1_body,
        out_type=(
            jax.ShapeDtypeStruct((B, KEEP), _i32),
            jax.ShapeDtypeStruct((NW * 16,), _i32),
        ),
        mesh=mesh,
        scratch_types=[
            pltpu.VMEM((N,), _f32),
            pltpu.VMEM((N,), _i32),
            pltpu.VMEM((8192,), _i32),
            pltpu.VMEM((CAND,), _i32),
            pltpu.VMEM((KEEP,), _i32),
            pltpu.VMEM((16,), _i32),
        ],
        compiler_params=params,
    )
    k2 = pl.kernel(
        _k2_body,
        out_type=jax.ShapeDtypeStruct((B, N), _f32),
        mesh=mesh,
        scratch_types=[
            pltpu.VMEM((RPS * KEEP,), _i32),
            pltpu.VMEM((N,), _f32),
            pltpu.VMEM((4096,), _i32),
            pltpu.VMEM((CAND,), _i32),
            pltpu.VMEM((256,), _i32),
            pltpu.VMEM((16,), _i32),
            pltpu.VMEM((256,), _i32),
            pltpu.VMEM((16,), _i32),
            pltpu.VMEM_SHARED((16, 4096), _i32),
            pltpu.VMEM_SHARED((256,), _i32),
        ],
        compiler_params=params,
    )
    return k1, k2


def kernel(x):
    k1, k2 = _build()
    kept, t = k1(x)
    return k2(x, kept.reshape(-1), t)


# speedup vs baseline: 1.2075x; 1.1049x over previous
import functools

import jax
import jax.numpy as jnp
from jax import lax
from jax.experimental import pallas as pl
from jax.experimental.pallas import tpu as pltpu
from jax.experimental.pallas import tpu_sc as plsc

B = 128
N = 32768
KP = 1639
KB = 104896
NC = 2
NS = 16
NW = NC * NS
RPW = B // NW
RPS = B // NS
KEEP = 2048
CAND = 16384
IMIN = -2147483648
M31 = 0x7FFFFFFF

_i32 = jnp.int32
_f32 = jnp.float32


def _scal(v):
    return v[0] if getattr(v, "ndim", 0) else v


def _sortable(v):
    bits = lax.bitcast_convert_type(v, _i32)
    return bits ^ ((bits >> 31) & M31)


def _crossing(vec, k, lane):
    r = jnp.flip(vec, 0)
    cum = plsc.cumsum(r)
    m = cum >= k
    j0 = _scal(plsc.all_reduce_ffs(m))
    sel = lane == j0
    cum_at = jnp.sum(jnp.where(sel, cum, 0))
    r_at = jnp.sum(jnp.where(sel, r, 0))
    u = 15 - j0
    k_new = k - (cum_at - r_at)
    return u, k_new, r_at


def _analyze_hist(hist_v, k, lane, base=0):
    zero16 = jnp.zeros((16,), _i32)

    def chunk_body(cj, cv):
        def inner(j, acc):
            return acc + hist_v[pl.ds(base + cj * 256 + j * 16, 16)]
        acc = lax.fori_loop(0, 16, inner, zero16)
        return jnp.where(lane == cj, jnp.sum(acc), cv)

    chunkvec = lax.fori_loop(0, 16, chunk_body, zero16)
    cstar, kc, _ = _crossing(chunkvec, k, lane)

    def bin_body(j, bv):
        w = hist_v[pl.ds(base + cstar * 256 + j * 16, 16)]
        return jnp.where(lane == j, jnp.sum(w), bv)

    binvec = lax.fori_loop(0, 16, bin_body, zero16)
    b_in, k_new, _ = _crossing(binvec, kc, lane)

    def zbody(j, _):
        hist_v[pl.ds(base + j * 16, 16)] = zero16
        return 0

    lax.fori_loop(0, 256, zbody, 0)
    return cstar * 16 + b_in, k_new


def _pc(mask):
    return _scal(plsc.all_reduce_population_count(mask))


def _compact(dst_ref, y, mask, off_v):
    pos = plsc.cumsum(jnp.where(mask, 1, 0))
    plsc.store_scatter(dst_ref, [off_v + pos - 1], y, mask=mask)
    return off_v + plsc.all_reduce_population_count(mask)


def _k1_body(x_hbm, kept_hbm, t_hbm, xrow_v, y_v, hist_v, cand_v, keep_v, tb_v):
    c = lax.axis_index("c")
    s = lax.axis_index("s")
    wid = s * NC + c
    lane = lax.iota(_i32, 16)
    zero16 = jnp.zeros((16,), _i32)
    ones16 = jnp.ones((16,), _i32)
    fill16 = jnp.full((16,), IMIN, _i32)

    def zh(j, _):
        hist_v[pl.ds(j * 16, 16)] = zero16
        return 0

    lax.fori_loop(0, 512, zh, 0)

    def row_body(r, carry):
        t_vec, bg = carry
        row = wid * RPW + r
        pltpu.sync_copy(x_hbm.at[row], xrow_v)

        def zk(j, _):
            keep_v[pl.ds(j * 16, 16)] = fill16
            return 0

        lax.fori_loop(0, KEEP // 16, zk, 0)
        bg_v = zero16 + bg

        def s1(i, carry):
            ko, co = carry
            for u in range(4):
                off = (i * 4 + u) * 16
                y = _sortable(xrow_v[pl.ds(off, 16)])
                y_v[pl.ds(off, 16)] = y
                b = ((y >> 24) + 128) & 0xFF
                plsc.addupdate_scatter(hist_v, [b * 16 + lane], ones16)
                mk = (b > bg_v) & (ko < KEEP - 15)
                ko = _compact(keep_v, y, mk, ko)
                mc = (b == bg_v) & (co < CAND - 15)
                idx = 4096 + ((y >> 16) & 0xFF) * 16 + lane
                plsc.addupdate_scatter(hist_v, [idx], ones16, mask=mc)
                co = _compact(cand_v, y, mc, co)
            return ko, co

        ko_v, n_v = lax.fori_loop(0, N // 64, s1, (zero16, zero16))
        b0, k1r = _analyze_hist(hist_v, _i32(KP), lane)

        miss = (b0 != bg).astype(_i32)
        b0_v = zero16 + b0

        def zk2(j, _):
            keep_v[pl.ds(j * 16, 16)] = fill16
            return 0

        lax.fori_loop(0, miss * (KEEP // 16), zk2, 0)

        def zh2(j, _):
            hist_v[pl.ds(4096 + j * 16, 16)] = zero16
            return 0

        lax.fori_loop(0, miss * 256, zh2, 0)

        def s2(i, carry):
            ko, co = carry
            for u in range(4):
                off = (i * 4 + u) * 16
                y = y_v[pl.ds(off, 16)]
                b = ((y >> 24) + 128) & 0xFF
                mk = (b > b0_v) & (ko < KEEP - 15)
                ko = _compact(keep_v, y, mk, ko)
                mc = (b == b0_v) & (co < CAND - 15)
                idx = 4096 + ((y >> 16) & 0xFF) * 16 + lane
                plsc.addupdate_scatter(hist_v, [idx], ones16, mask=mc)
                co = _compact(cand_v, y, mc, co)
            return ko, co

        ko_r, n_r = lax.fori_loop(0, miss * (N // 64), s2, (zero16, zero16))
        ko_v = jnp.where(miss > 0, ko_r, ko_v)
        n_v = jnp.where(miss > 0, n_r, n_v)
        t_p = (b0 - 128) << 24

        def lvl(l, carry):
            ko, n_v, k_rem, t_p = carry
            sh = 24 - 8 * l
            b_l, k_new = _analyze_hist(hist_v, k_rem, lane, base=4096 * (l & 1))
            b_l_v = zero16 + b_l
            obase = 4096 * ((l + 1) & 1)

            def scn(i, cy):
                ko2, co2 = cy
                off = i * 16
                valid = (off + lane) < n_v
                y = cand_v[pl.ds(off, 16)]
                byt = (y >> sh) & 0xFF
                mk = valid & (byt > b_l_v) & (ko2 < KEEP - 15)
                ko2 = _compact(keep_v, y, mk, ko2)
                mc = valid & (byt == b_l_v)
                idx = obase + ((y >> (sh - 8)) & 0xFF) * 16 + lane
                plsc.addupdate_scatter(hist_v, [idx], ones16, mask=mc)
                co2 = _compact(cand_v, y, mc, co2)
                return ko2, co2

            ko, n2_v = lax.fori_loop(0, (n_v[0] + 15) >> 4, scn, (ko, zero16))
            t_p = t_p | (b_l << sh)
            return ko, n2_v, k_new, t_p

        ko_v, n3_v, k3, t_p = lax.fori_loop(1, 3, lvl, (ko_v, n_v, k1r, t_p))

        b3, _ = _analyze_hist(hist_v, k3, lane, base=4096)
        t_row = t_p | b3
        b3_v = zero16 + b3

        def fin(i, ko2):
            off = i * 16
            valid = (off + lane) < n3_v
            y = cand_v[pl.ds(off, 16)]
            mk = valid & ((y & 0xFF) >= b3_v) & (ko2 < KEEP - 15)
            return _compact(keep_v, y, mk, ko2)

        lax.fori_loop(0, (n3_v[0] + 15) >> 4, fin, ko_v)
        pltpu.sync_copy(keep_v, kept_hbm.at[row])
        return jnp.where(lane == r, t_row, t_vec), b0

    t_vec, _ = lax.fori_loop(0, RPW, row_body,
                             (jnp.zeros((16,), _i32), _i32(191)))
    tb_v[...] = t_vec
    pltpu.sync_copy(tb_v, t_hbm.at[pl.ds(wid * 16, 16)])


def _share_analyze(k, s, lane, hist_v, tmp_v, st_v, cnt_v, sh_hist, sh_cnt):
    zero16 = jnp.zeros((16,), _i32)
    pltpu.sync_copy(hist_v, sh_hist.at[s])

    def zh(j, _):
        hist_v[pl.ds(j * 16, 16)] = zero16
        return 0

    lax.fori_loop(0, 256, zh, 0)
    plsc.subcore_barrier()

    def wbody(w, accs):
        pltpu.sync_copy(sh_hist.at[w, pl.ds(s * 256, 256)], tmp_v)
        return tuple(accs[j] + tmp_v[pl.ds(j * 16, 16)] for j in range(16))

    accs = lax.fori_loop(0, 16, wbody, (zero16,) * 16)
    binvec = zero16
    for j in range(16):
        binvec = jnp.where(lane == j, jnp.sum(accs[j]), binvec)
    st_v[...] = binvec
    pltpu.sync_copy(st_v, sh_cnt.at[pl.ds(s * 16, 16)])
    plsc.subcore_barrier()
    pltpu.sync_copy(sh_cnt, cnt_v)

    def cb(cj, cv):
        return jnp.where(lane == cj, jnp.sum(cnt_v[pl.ds(cj * 16, 16)]), cv)

    chunkvec = lax.fori_loop(0, 16, cb, zero16)
    cstar, kc, _ = _crossing(chunkvec, k, lane)
    binvec2 = cnt_v[pl.ds(cstar * 16, 16)]
    b_in, k_new, _ = _crossing(binvec2, kc, lane)
    return cstar * 16 + b_in, k_new


def _k2_body(x_hbm, kept_hbm, t_hbm, out_hbm,
             kv_v, xrow_v, hist_v, cand_v, tmp_v, st_v, cnt_v, t16_v,
             sh_hist, sh_cnt):
    c = lax.axis_index("c")
    s = lax.axis_index("s")
    wid = s * NC + c
    lane = lax.iota(_i32, 16)
    zero16 = jnp.zeros((16,), _i32)
    ones16 = jnp.ones((16,), _i32)
    NKV = RPS * KEEP

    def zh(j, _):
        hist_v[pl.ds(j * 16, 16)] = zero16
        return 0

    lax.fori_loop(0, 256, zh, 0)
    pltpu.sync_copy(kept_hbm.at[pl.ds(s * NKV, NKV)], kv_v)

    def h0(i, _):
        for u in range(4):
            off = (i * 4 + u) * 16
            y = kv_v[pl.ds(off, 16)]
            b = ((y >> 24) + 128) & 0xFF
            plsc.addupdate_scatter(hist_v, [b * 16 + lane], ones16)
        return 0

    lax.fori_loop(0, NKV // 64, h0, 0)
    b0, k = _share_analyze(_i32(KB), s, lane, hist_v, tmp_v, st_v, cnt_v,
                           sh_hist, sh_cnt)

    b0_v = zero16 + b0

    def c0(i, co):
        for u in range(4):
            off = (i * 4 + u) * 16
            y = kv_v[pl.ds(off, 16)]
            b = ((y >> 24) + 128) & 0xFF
            mc = (b == b0_v) & (co < CAND - 15)
            idx = ((y >> 16) & 0xFF) * 16 + lane
            plsc.addupdate_scatter(hist_v, [idx], ones16, mask=mc)
            co = _compact(cand_v, y, mc, co)
        return co

    n_v = lax.fori_loop(0, NKV // 64, c0, zero16)
    T = (b0 - 128) << 24

    for l in (1, 2):
        sh = 24 - 8 * l
        b_l, k = _share_analyze(k, s, lane, hist_v, tmp_v, st_v, cnt_v,
                                sh_hist, sh_cnt)

        b_l_v = zero16 + b_l

        def cl(i, co, n_v=n_v, b_l_v=b_l_v, sh=sh):
            off = i * 16
            valid = (off + lane) < n_v
            y = cand_v[pl.ds(off, 16)]
            byt = (y >> sh) & 0xFF
            mc = valid & (byt == b_l_v) & (co < CAND - 15)
            idx = ((y >> (sh - 8)) & 0xFF) * 16 + lane
            plsc.addupdate_scatter(hist_v, [idx], ones16, mask=mc)
            return _compact(cand_v, y, mc, co)

        n_v = lax.fori_loop(0, (n_v[0] + 15) >> 4, cl, zero16)
        T = T | (b_l << sh)

    b3, k = _share_analyze(k, s, lane, hist_v, tmp_v, st_v, cnt_v,
                           sh_hist, sh_cnt)
    T = T | b3

    pltpu.sync_copy(t_hbm.at[pl.ds(wid * 16, 16)], t16_v)
    thr_vec = jnp.maximum(t16_v[...], T)

    def mrow(r, _):
        row = wid * RPW + r
        pltpu.sync_copy(x_hbm.at[row], xrow_v)
        thr = jnp.max(jnp.where(lane == r, thr_vec, IMIN))

        def mb(i, _):
            for u in range(4):
                off = (i * 4 + u) * 16
                v = xrow_v[pl.ds(off, 16)]
                keep = _sortable(v) >= thr
                xrow_v[pl.ds(off, 16)] = jnp.where(keep, v, _f32(0.0))
            return 0

        lax.fori_loop(0, N // 64, mb, 0)
        pltpu.sync_copy(xrow_v, out_hbm.at[row])
        return 0

    lax.fori_loop(0, RPW, mrow, 0)


@functools.cache
def _build():
    mesh = plsc.VectorSubcoreMesh(core_axis_name="c", subcore_axis_name="s")
    params = pltpu.CompilerParams(needs_layout_passes=False)
    k1 = pl.kernel(
        _k1_body,
        out_type=(
            jax.ShapeDtypeStruct((B, KEEP), _i32),
            jax.ShapeDtypeStruct((NW * 16,), _i32),
        ),
        mesh=mesh,
        scratch_types=[
            pltpu.VMEM((N,), _f32),
            pltpu.VMEM((N,), _i32),
            pltpu.VMEM((8192,), _i32),
            pltpu.VMEM((CAND,), _i32),
            pltpu.VMEM((KEEP,), _i32),
            pltpu.VMEM((16,), _i32),
        ],
        compiler_params=params,
    )
    k2 = pl.kernel(
        _k2_body,
        out_type=jax.ShapeDtypeStruct((B, N), _f32),
        mesh=mesh,
        scratch_types=[
            pltpu.VMEM((RPS * KEEP,), _i32),
            pltpu.VMEM((N,), _f32),
            pltpu.VMEM((4096,), _i32),
            pltpu.VMEM((CAND,), _i32),
            pltpu.VMEM((256,), _i32),
            pltpu.VMEM((16,), _i32),
            pltpu.VMEM((256,), _i32),
            pltpu.VMEM((16,), _i32),
            pltpu.VMEM_SHARED((16, 4096), _i32),
            pltpu.VMEM_SHARED((256,), _i32),
        ],
        compiler_params=params,
    )
    return k1, k2


def kernel(x):
    k1, k2 = _build()
    kept, t = k1(x)
    return k2(x, kept.reshape(-1), t)
